# speedup vs baseline: 22.4546x; 22.4546x over previous
import jax
import jax.numpy as jnp
from jax.experimental import pallas as pl
from jax.experimental.pallas import tpu as pltpu

B = 8
D = 512
ML = 256
T = 2 * ML - 1


def _splitnet_kernel(x_ref, w_ref, lnw_ref, lnb_ref, lb_ref, slv_ref,
                     sls_ref, lab_ref,
                     leaf_ref, sc_ref,
                     tree_ref, stack_ref, scal_ref):
    leaf_ref[...] = jnp.zeros((B, ML, D), jnp.float32)
    tree_ref[:, 0:1, :] = x_ref[...].reshape(B, 1, D)
    for b in range(B):
        stack_ref[b, 0] = 0
        scal_ref[0, b] = 1
        scal_ref[1, b] = 1
        scal_ref[2, b] = 0

    steps = sls_ref[0, 0]
    for b in range(1, B):
        steps = jnp.maximum(steps, sls_ref[0, b])

    def step(i, scores):
        parts = []
        for b in range(B):
            act = i < sls_ref[0, b]
            sp_b = scal_ref[0, b]
            pop = jnp.where(act, sp_b - 1, 0)
            nid = stack_ref[b, pop]
            scal_ref[0, b] = jnp.where(act, sp_b - 1, sp_b)
            parts.append(tree_ref[b, pl.ds(nid, 1), :])
        parent = jnp.concatenate(parts, axis=0)

        mu = jnp.mean(parent, axis=1, keepdims=True)
        cen = parent - mu
        var = jnp.mean(cen * cen, axis=1, keepdims=True)
        xn = cen * jax.lax.rsqrt(var + 1e-5) * lnw_ref[...] + lnb_ref[...]
        y = jax.lax.dot_general(xn, w_ref[...], (((1,), (1,)), ((), ())),
                                preferred_element_type=jnp.float32)
        gate = jax.nn.sigmoid(y + lb_ref[...])
        left = gate * parent
        right = (1.0 - gate) * parent

        num = jnp.sum(left * right, axis=1, keepdims=True)
        na = jnp.maximum(jnp.sqrt(jnp.sum(left * left, axis=1, keepdims=True)), 1e-8)
        nb = jnp.maximum(jnp.sqrt(jnp.sum(right * right, axis=1, keepdims=True)), 1e-8)
        s = num / (na * nb)

        act_v = slv_ref[...] > i
        col = jax.lax.broadcasted_iota(jnp.int32, (B, D), 1)
        scores = scores + jnp.where((col == i) & act_v, s, 0.0)

        for b in range(B):
            act = i < sls_ref[0, b]
            split = jnp.logical_and(act, lab_ref[b, i] > 0)
            sp_b = scal_ref[0, b]
            nl_b = scal_ref[1, b]
            lc_b = scal_ref[2, b]

            @pl.when(split)
            def _(b=b, sp_b=sp_b, nl_b=nl_b):
                tree_ref[b, pl.ds(nl_b, 1), :] = left[b:b + 1, :]
                tree_ref[b, pl.ds(nl_b + 1, 1), :] = right[b:b + 1, :]
                stack_ref[b, sp_b] = nl_b + 1
                stack_ref[b, sp_b + 1] = nl_b
                scal_ref[0, b] = sp_b + 2
                scal_ref[1, b] = nl_b + 2

            @pl.when(jnp.logical_and(act, jnp.logical_not(split)))
            def _(b=b, lc_b=lc_b):
                leaf_ref[b, pl.ds(lc_b, 1), :] = parent[b:b + 1, :]
                scal_ref[2, b] = lc_b + 1
        return scores

    scores = jax.lax.fori_loop(0, steps, step, jnp.zeros((B, D), jnp.float32))
    sc_ref[...] = scores


def kernel(input_, features, length, label, ln_weight, ln_bias, lin_weight, lin_bias):
    del features
    length = length.astype(jnp.int32)
    label = label.astype(jnp.int32)
    sl = 2 * length - 1

    leaf, scores = pl.pallas_call(
        _splitnet_kernel,
        out_shape=[
            jax.ShapeDtypeStruct((B, ML, D), jnp.float32),
            jax.ShapeDtypeStruct((B, D), jnp.float32),
        ],
        in_specs=[
            pl.BlockSpec(memory_space=pltpu.VMEM),
            pl.BlockSpec(memory_space=pltpu.VMEM),
            pl.BlockSpec(memory_space=pltpu.VMEM),
            pl.BlockSpec(memory_space=pltpu.VMEM),
            pl.BlockSpec(memory_space=pltpu.VMEM),
            pl.BlockSpec(memory_space=pltpu.VMEM),
            pl.BlockSpec(memory_space=pltpu.SMEM),
            pl.BlockSpec(memory_space=pltpu.SMEM),
        ],
        out_specs=[
            pl.BlockSpec(memory_space=pltpu.VMEM),
            pl.BlockSpec(memory_space=pltpu.VMEM),
        ],
        scratch_shapes=[
            pltpu.VMEM((B, D, D), jnp.float32),
            pltpu.SMEM((B, D), jnp.int32),
            pltpu.SMEM((4, B), jnp.int32),
        ],
    )(
        input_,
        lin_weight,
        ln_weight.reshape(1, D),
        ln_bias.reshape(1, D),
        lin_bias.reshape(1, D),
        sl.reshape(B, 1),
        sl.reshape(1, B),
        label[:, :T],
    )
    return leaf, scores[:, :T]
